# Initial kernel scaffold; baseline (speedup 1.0000x reference)
import jax
import jax.numpy as jnp
from jax.experimental import pallas as pl


def kernel(data):
    raise NotImplementedError("write your pallas kernel here")


# speedup vs baseline: 58.8185x; 58.8185x over previous
import functools
import math

import ml_dtypes
import numpy as np
import jax
import jax.numpy as jnp
from jax import lax
from jax.experimental import pallas as pl
from jax.experimental.pallas import tpu as pltpu
from jax.experimental.pallas import tpu_sc as plsc

IMG = 256
NPIX = IMG * IMG
BATCH = 8
NPTS = 2048
TOT = BATCH * NPTS
NWORK = 32
PPW = TOT // NWORK
WIN = 22
NCELL = WIN * WIN
NCHUNK = (NCELL + 15) // 16
CPAD = NCHUNK * 16


def _build_mat():
    def normalize(x):
        n = np.linalg.norm(x, axis=1, keepdims=True).astype(np.float32)
        return x / np.maximum(n, np.float32(1e-6))

    eyes = np.array([[-1.0, -1.0, -1.0]], np.float32)
    centers = np.zeros((1, 3), np.float32)
    ups = np.array([[0.0, 0.0, 1.0]], np.float32)
    zaxis = normalize(eyes - centers)
    xaxis = normalize(np.cross(ups, zaxis).astype(np.float32))
    yaxis = np.cross(zaxis, xaxis).astype(np.float32)
    translation = np.eye(4, dtype=np.float32)
    translation[:3, 3] = -eyes[0]
    orientation = np.eye(4, dtype=np.float32)
    orientation[0, :3] = xaxis[0]
    orientation[1, :3] = yaxis[0]
    orientation[2, :3] = zaxis[0]
    view = orientation @ translation
    z_near, z_far = np.float32(0.1), np.float32(10.0)
    proj = np.zeros((4, 4), np.float32)
    proj[0, 0] = 1.5
    proj[1, 1] = 1.5
    proj[2, 2] = -2.0 / (z_far - z_near)
    proj[2, 3] = (z_far + z_near) / (z_far - z_near)
    proj[3, 3] = 1.0
    return (proj @ view).astype(np.float32)


_MAT = _build_mat()
_MAT_BF = _MAT.astype(ml_dtypes.bfloat16).astype(np.float32)

_A2 = (math.pi / 20.0) ** 2
_COEFS = tuple(
    (-1.0) ** k * _A2 ** k / math.factorial(2 * k) for k in range(7)
)

_offs = np.arange(-10, 12)
_DI = np.repeat(_offs, WIN)
_DJ = np.tile(_offs, WIN)
_pad = np.full(CPAD - NCELL, 10000)
_DI = np.concatenate([_DI, _pad])
_DJ = np.concatenate([_DJ, _pad])
_DIF_NP = _DI.astype(np.float32)
_DJF_NP = _DJ.astype(np.float32)
_DII_NP = _DI.astype(np.int32)
_DJI_NP = _DJ.astype(np.int32)


def _proj_body(d_ref, bi_ref, bj_ref, fi_ref, fj_ref, ft_ref):
    x = d_ref[0:1, :].astype(jnp.bfloat16).astype(jnp.float32)
    y = d_ref[1:2, :].astype(jnp.bfloat16).astype(jnp.float32)
    z = d_ref[2:3, :].astype(jnp.bfloat16).astype(jnp.float32)
    m = _MAT_BF
    pos_x = m[0, 0] * x + m[0, 1] * y + m[0, 2] * z + m[0, 3]
    pos_y = m[1, 0] * x + m[1, 1] * y + m[1, 2] * z + m[1, 3]
    pos_z = m[2, 0] * x + m[2, 1] * y + m[2, 2] * z + m[2, 3]
    pts_i = (-pos_y + 1.0) / 2.0 * (IMG - 1.0)
    pts_j = (pos_x + 1.0) / 2.0 * (IMG - 1.0)
    zmin = jnp.min(pos_z)
    zmax = jnp.max(pos_z)
    feat = 1.0 - (pos_z - zmin) / (zmax - zmin)
    base_i = jnp.floor(pts_i)
    base_j = jnp.floor(pts_j)
    fi_ref[...] = pts_i - base_i
    fj_ref[...] = pts_j - base_j
    ft_ref[...] = feat
    bi_ref[...] = base_i.astype(jnp.int32)
    bj_ref[...] = base_j.astype(jnp.int32)


_proj = pl.pallas_call(
    _proj_body,
    out_shape=[
        jax.ShapeDtypeStruct((1, TOT), jnp.int32),
        jax.ShapeDtypeStruct((1, TOT), jnp.int32),
        jax.ShapeDtypeStruct((1, TOT), jnp.float32),
        jax.ShapeDtypeStruct((1, TOT), jnp.float32),
        jax.ShapeDtypeStruct((1, TOT), jnp.float32),
    ],
)


def _splat_body(bi_h, bj_h, fi_h, fj_h, ft_h, dif_h, djf_h, dii_h, dji_h,
                out_h, img, bi, bj, fi, fj, ft, cbi, cbj, cfi, cfj, cft,
                dif, djf, dii, dji):
    wid = lax.axis_index("s") * 2 + lax.axis_index("c")
    base = wid * PPW

    pltpu.sync_copy(bi_h.at[pl.ds(base, PPW)], bi)
    pltpu.sync_copy(bj_h.at[pl.ds(base, PPW)], bj)
    pltpu.sync_copy(fi_h.at[pl.ds(base, PPW)], fi)
    pltpu.sync_copy(fj_h.at[pl.ds(base, PPW)], fj)
    pltpu.sync_copy(ft_h.at[pl.ds(base, PPW)], ft)
    pltpu.sync_copy(dif_h, dif)
    pltpu.sync_copy(djf_h, djf)
    pltpu.sync_copy(dii_h, dii)
    pltpu.sync_copy(dji_h, dji)

    zeros16 = jnp.zeros((16,), jnp.float32)
    zidx = jnp.zeros((16,), jnp.int32)

    def zbody(k, c):
        b0 = k * 128
        for u in range(8):
            img[pl.ds(b0 + u * 16, 16)] = zeros16
        return c

    lax.fori_loop(0, NPIX // 128, zbody, 0)

    nvec = jnp.zeros((16,), jnp.int32)
    for c in range(PPW // 16):
        s = c * 16
        bic = bi[pl.ds(s, 16)]
        bjc = bj[pl.ds(s, 16)]
        act = ((bic >= -10) & (bic <= IMG + 9)
               & (bjc >= -10) & (bjc <= IMG + 9))
        acti = act.astype(jnp.int32)
        dst = nvec + plsc.cumsum(acti) - acti
        plsc.store_scatter(cbi, [dst], bic, mask=act)
        plsc.store_scatter(cbj, [dst], bjc, mask=act)
        plsc.store_scatter(cfi, [dst], fi[pl.ds(s, 16)], mask=act)
        plsc.store_scatter(cfj, [dst], fj[pl.ds(s, 16)], mask=act)
        plsc.store_scatter(cft, [dst], ft[pl.ds(s, 16)], mask=act)
        nvec = nvec + plsc.all_reduce_population_count(act)
    n = jnp.max(nvec)

    def pbody(p, carry):
        bip = cbi[pl.ds(p, 16)][zidx]
        bjp = cbj[pl.ds(p, 16)][zidx]
        fip = cfi[pl.ds(p, 16)][zidx]
        fjp = cfj[pl.ds(p, 16)][zidx]
        ftp = cft[pl.ds(p, 16)][zidx]
        for c in range(NCHUNK):
            s = c * 16
            ai = dif[pl.ds(s, 16)] - fip
            aj = djf[pl.ds(s, 16)] - fjp
            d2 = ai * ai + aj * aj
            w = jnp.float32(_COEFS[-1])
            for cf in _COEFS[-2::-1]:
                w = w * d2 + jnp.float32(cf)
            val = w * ftp
            ii = bip + dii[pl.ds(s, 16)]
            jj = bjp + dji[pl.ds(s, 16)]
            msk = ((ii >= 0) & (ii <= IMG - 1) & (jj >= 0) & (jj <= IMG - 1)
                   & (d2 <= 100.0))
            idx = ((ii << 8) + jj) & (NPIX - 1)
            old = plsc.load_gather(img, [idx])
            plsc.store_scatter(img, [idx], jnp.maximum(old, val), mask=msk)
        return carry

    lax.fori_loop(0, n, pbody, 0)

    pltpu.sync_copy(img, out_h.at[wid])


@functools.lru_cache(maxsize=None)
def _make_splat():
    return functools.partial(
        pl.kernel,
        out_type=jax.ShapeDtypeStruct((NWORK, NPIX), jnp.float32),
        mesh=plsc.VectorSubcoreMesh(core_axis_name="c", subcore_axis_name="s",
                                    num_cores=2, num_subcores=16),
        scratch_types=[
            pltpu.VMEM((NPIX,), jnp.float32),
            pltpu.VMEM((PPW,), jnp.int32),
            pltpu.VMEM((PPW,), jnp.int32),
            pltpu.VMEM((PPW,), jnp.float32),
            pltpu.VMEM((PPW,), jnp.float32),
            pltpu.VMEM((PPW,), jnp.float32),
            pltpu.VMEM((PPW + 16,), jnp.int32),
            pltpu.VMEM((PPW + 16,), jnp.int32),
            pltpu.VMEM((PPW + 16,), jnp.float32),
            pltpu.VMEM((PPW + 16,), jnp.float32),
            pltpu.VMEM((PPW + 16,), jnp.float32),
            pltpu.VMEM((CPAD,), jnp.float32),
            pltpu.VMEM((CPAD,), jnp.float32),
            pltpu.VMEM((CPAD,), jnp.int32),
            pltpu.VMEM((CPAD,), jnp.int32),
        ],
        compiler_params=pltpu.CompilerParams(needs_layout_passes=False),
    )(_splat_body)


def _merge_body(a_ref, b_ref, c_ref, d_ref, o_ref):
    o_ref[...] = jnp.maximum(jnp.maximum(a_ref[...], b_ref[...]),
                             jnp.maximum(c_ref[...], d_ref[...]))


def _make_merge():
    def imap(k):
        return lambda b: (4 * b + k, 0, 0)

    return pl.pallas_call(
        _merge_body,
        grid=(BATCH,),
        in_specs=[pl.BlockSpec((1, 1, NPIX), imap(k)) for k in range(4)],
        out_specs=pl.BlockSpec((1, 1, NPIX), lambda b: (b, 0, 0)),
        out_shape=jax.ShapeDtypeStruct((BATCH, 1, NPIX), jnp.float32),
    )


_merge = _make_merge()


def kernel(data):
    d = data.reshape(TOT, 3).T
    d = jnp.pad(d, ((0, 5), (0, 0)))
    bi, bj, fi, fj, ft = _proj(d)
    bi = bi.reshape(TOT)
    bj = bj.reshape(TOT)
    fi = fi.reshape(TOT)
    fj = fj.reshape(TOT)
    ft = ft.reshape(TOT)
    copies = _make_splat()(bi, bj, fi, fj, ft,
                    jnp.asarray(_DIF_NP), jnp.asarray(_DJF_NP),
                    jnp.asarray(_DII_NP), jnp.asarray(_DJI_NP))
    c3 = copies.reshape(NWORK, 1, NPIX)
    img = _merge(c3, c3, c3, c3)
    return img.reshape(BATCH, 1, IMG, IMG)


# speedup vs baseline: 119.1630x; 2.0259x over previous
import functools
import math

import ml_dtypes
import numpy as np
import jax
import jax.numpy as jnp
from jax import lax
from jax.experimental import pallas as pl
from jax.experimental.pallas import tpu as pltpu
from jax.experimental.pallas import tpu_sc as plsc

IMG = 256
NPIX = IMG * IMG
BATCH = 8
NPTS = 2048
TOT = BATCH * NPTS
NWORK = 32
PPW = TOT // NWORK
WIN = 22


def _build_mat():
    def normalize(x):
        n = np.linalg.norm(x, axis=1, keepdims=True).astype(np.float32)
        return x / np.maximum(n, np.float32(1e-6))

    eyes = np.array([[-1.0, -1.0, -1.0]], np.float32)
    centers = np.zeros((1, 3), np.float32)
    ups = np.array([[0.0, 0.0, 1.0]], np.float32)
    zaxis = normalize(eyes - centers)
    xaxis = normalize(np.cross(ups, zaxis).astype(np.float32))
    yaxis = np.cross(zaxis, xaxis).astype(np.float32)
    translation = np.eye(4, dtype=np.float32)
    translation[:3, 3] = -eyes[0]
    orientation = np.eye(4, dtype=np.float32)
    orientation[0, :3] = xaxis[0]
    orientation[1, :3] = yaxis[0]
    orientation[2, :3] = zaxis[0]
    view = orientation @ translation
    z_near, z_far = np.float32(0.1), np.float32(10.0)
    proj = np.zeros((4, 4), np.float32)
    proj[0, 0] = 1.5
    proj[1, 1] = 1.5
    proj[2, 2] = -2.0 / (z_far - z_near)
    proj[2, 3] = (z_far + z_near) / (z_far - z_near)
    proj[3, 3] = 1.0
    return (proj @ view).astype(np.float32)


_MAT = _build_mat()
_MAT_BF = _MAT.astype(ml_dtypes.bfloat16).astype(np.float32)

_A2 = (math.pi / 20.0) ** 2
_COEFS = tuple(
    (-1.0) ** k * _A2 ** k / math.factorial(2 * k) for k in range(7)
)

_offs = np.arange(-10, 12)
_DI_g, _DJ_g = np.meshgrid(_offs, _offs, indexing="ij")
_mind = np.where(_offs <= 0, -_offs, _offs - 1)
_MI_g, _MJ_g = np.meshgrid(_mind, _mind, indexing="ij")
_keep = (_MI_g ** 2 + _MJ_g ** 2) <= 100
_DI = _DI_g[_keep]
_DJ = _DJ_g[_keep]
NCELLC = _DI.size
NCHUNK = (NCELLC + 15) // 16
CPAD = NCHUNK * 16
_pad = np.full(CPAD - NCELLC, 10000)
_DI = np.concatenate([_DI, _pad])
_DJ = np.concatenate([_DJ, _pad])
_DIF_NP = _DI.astype(np.float32)
_DJF_NP = _DJ.astype(np.float32)
_DII_NP = _DI.astype(np.int32)
_DJI_NP = _DJ.astype(np.int32)
_DIDX_NP = np.where(_DI >= 1000, 0, _DI * IMG + _DJ).astype(np.int32)


def _proj_body(d_ref, bi_ref, bj_ref, fi_ref, fj_ref, ft_ref):
    x = d_ref[0:1, :].astype(jnp.bfloat16).astype(jnp.float32)
    y = d_ref[1:2, :].astype(jnp.bfloat16).astype(jnp.float32)
    z = d_ref[2:3, :].astype(jnp.bfloat16).astype(jnp.float32)
    m = _MAT_BF
    pos_x = m[0, 0] * x + m[0, 1] * y + m[0, 2] * z + m[0, 3]
    pos_y = m[1, 0] * x + m[1, 1] * y + m[1, 2] * z + m[1, 3]
    pos_z = m[2, 0] * x + m[2, 1] * y + m[2, 2] * z + m[2, 3]
    pts_i = (-pos_y + 1.0) / 2.0 * (IMG - 1.0)
    pts_j = (pos_x + 1.0) / 2.0 * (IMG - 1.0)
    zmin = jnp.min(pos_z)
    zmax = jnp.max(pos_z)
    feat = 1.0 - (pos_z - zmin) / (zmax - zmin)
    base_i = jnp.floor(pts_i)
    base_j = jnp.floor(pts_j)
    fi_ref[...] = pts_i - base_i
    fj_ref[...] = pts_j - base_j
    ft_ref[...] = feat
    bi_ref[...] = base_i.astype(jnp.int32)
    bj_ref[...] = base_j.astype(jnp.int32)


_proj = pl.pallas_call(
    _proj_body,
    out_shape=[
        jax.ShapeDtypeStruct((1, TOT), jnp.int32),
        jax.ShapeDtypeStruct((1, TOT), jnp.int32),
        jax.ShapeDtypeStruct((1, TOT), jnp.float32),
        jax.ShapeDtypeStruct((1, TOT), jnp.float32),
        jax.ShapeDtypeStruct((1, TOT), jnp.float32),
    ],
)


def _splat_body(bi_h, bj_h, fi_h, fj_h, ft_h, dif_h, djf_h, dii_h, dji_h,
                didx_h, out_h, img, bi, bj, fi, fj, ft,
                ibi, ibj, ifi, ifj, ift, bbi, bbj, bfi, bfj, bft,
                dif, djf, dii, dji, didx):
    wid = lax.axis_index("s") * 2 + lax.axis_index("c")
    base = wid * PPW

    pltpu.sync_copy(bi_h.at[pl.ds(base, PPW)], bi)
    pltpu.sync_copy(bj_h.at[pl.ds(base, PPW)], bj)
    pltpu.sync_copy(fi_h.at[pl.ds(base, PPW)], fi)
    pltpu.sync_copy(fj_h.at[pl.ds(base, PPW)], fj)
    pltpu.sync_copy(ft_h.at[pl.ds(base, PPW)], ft)
    pltpu.sync_copy(dif_h, dif)
    pltpu.sync_copy(djf_h, djf)
    pltpu.sync_copy(dii_h, dii)
    pltpu.sync_copy(dji_h, dji)
    pltpu.sync_copy(didx_h, didx)

    zeros16 = jnp.zeros((16,), jnp.float32)
    zidx = jnp.zeros((16,), jnp.int32)

    def zbody(k, c):
        b0 = k * 128
        for u in range(8):
            img[pl.ds(b0 + u * 16, 16)] = zeros16
        return c

    lax.fori_loop(0, NPIX // 128, zbody, 0)

    nivec = jnp.zeros((16,), jnp.int32)
    nbvec = jnp.zeros((16,), jnp.int32)
    for c in range(PPW // 16):
        s = c * 16
        bic = bi[pl.ds(s, 16)]
        bjc = bj[pl.ds(s, 16)]
        fic = fi[pl.ds(s, 16)]
        fjc = fj[pl.ds(s, 16)]
        ftc = ft[pl.ds(s, 16)]
        act = ((bic >= -10) & (bic <= IMG + 9)
               & (bjc >= -10) & (bjc <= IMG + 9))
        inter = ((bic >= 10) & (bic <= IMG - 12)
                 & (bjc >= 10) & (bjc <= IMG - 12))
        bord = act & (~inter)
        ini = inter.astype(jnp.int32)
        boi = bord.astype(jnp.int32)
        dsti = nivec + plsc.cumsum(ini) - ini
        dstb = nbvec + plsc.cumsum(boi) - boi
        plsc.store_scatter(ibi, [dsti], bic, mask=inter)
        plsc.store_scatter(ibj, [dsti], bjc, mask=inter)
        plsc.store_scatter(ifi, [dsti], fic, mask=inter)
        plsc.store_scatter(ifj, [dsti], fjc, mask=inter)
        plsc.store_scatter(ift, [dsti], ftc, mask=inter)
        plsc.store_scatter(bbi, [dstb], bic, mask=bord)
        plsc.store_scatter(bbj, [dstb], bjc, mask=bord)
        plsc.store_scatter(bfi, [dstb], fic, mask=bord)
        plsc.store_scatter(bfj, [dstb], fjc, mask=bord)
        plsc.store_scatter(bft, [dstb], ftc, mask=bord)
        nivec = nivec + plsc.all_reduce_population_count(inter)
        nbvec = nbvec + plsc.all_reduce_population_count(bord)
    ni = jnp.max(nivec)
    nb = jnp.max(nbvec)

    def ibody(p, carry):
        bip = ibi[pl.ds(p, 16)][zidx]
        bjp = ibj[pl.ds(p, 16)][zidx]
        fip = ifi[pl.ds(p, 16)][zidx]
        fjp = ifj[pl.ds(p, 16)][zidx]
        ftp = ift[pl.ds(p, 16)][zidx]
        pbase = (bip << 8) + bjp

        @plsc.parallel_loop(0, CPAD, 16)
        def chunk(s):
            ai = dif[pl.ds(s, 16)] - fip
            aj = djf[pl.ds(s, 16)] - fjp
            d2 = ai * ai + aj * aj
            w = jnp.float32(_COEFS[-1])
            for cf in _COEFS[-2::-1]:
                w = w * d2 + jnp.float32(cf)
            val = w * ftp
            idx = pbase + didx[pl.ds(s, 16)]
            msk = d2 <= 100.0
            old = plsc.load_gather(img, [idx])
            plsc.store_scatter(img, [idx], jnp.maximum(old, val), mask=msk)

        return carry

    lax.fori_loop(0, ni, ibody, 0)

    def bbody(p, carry):
        bip = bbi[pl.ds(p, 16)][zidx]
        bjp = bbj[pl.ds(p, 16)][zidx]
        fip = bfi[pl.ds(p, 16)][zidx]
        fjp = bfj[pl.ds(p, 16)][zidx]
        ftp = bft[pl.ds(p, 16)][zidx]

        @plsc.parallel_loop(0, CPAD, 16)
        def chunk(s):
            ai = dif[pl.ds(s, 16)] - fip
            aj = djf[pl.ds(s, 16)] - fjp
            d2 = ai * ai + aj * aj
            w = jnp.float32(_COEFS[-1])
            for cf in _COEFS[-2::-1]:
                w = w * d2 + jnp.float32(cf)
            val = w * ftp
            ii = bip + dii[pl.ds(s, 16)]
            jj = bjp + dji[pl.ds(s, 16)]
            msk = ((ii >= 0) & (ii <= IMG - 1) & (jj >= 0) & (jj <= IMG - 1)
                   & (d2 <= 100.0))
            idx = ((ii << 8) + jj) & (NPIX - 1)
            old = plsc.load_gather(img, [idx])
            plsc.store_scatter(img, [idx], jnp.maximum(old, val), mask=msk)

        return carry

    lax.fori_loop(0, nb, bbody, 0)

    pltpu.sync_copy(img, out_h.at[wid])


@functools.lru_cache(maxsize=None)
def _make_splat():
    return functools.partial(
        pl.kernel,
        out_type=jax.ShapeDtypeStruct((NWORK, NPIX), jnp.float32),
        mesh=plsc.VectorSubcoreMesh(core_axis_name="c", subcore_axis_name="s",
                                    num_cores=2, num_subcores=16),
        scratch_types=[
            pltpu.VMEM((NPIX,), jnp.float32),
            pltpu.VMEM((PPW,), jnp.int32),
            pltpu.VMEM((PPW,), jnp.int32),
            pltpu.VMEM((PPW,), jnp.float32),
            pltpu.VMEM((PPW,), jnp.float32),
            pltpu.VMEM((PPW,), jnp.float32),
            pltpu.VMEM((PPW + 16,), jnp.int32),
            pltpu.VMEM((PPW + 16,), jnp.int32),
            pltpu.VMEM((PPW + 16,), jnp.float32),
            pltpu.VMEM((PPW + 16,), jnp.float32),
            pltpu.VMEM((PPW + 16,), jnp.float32),
            pltpu.VMEM((PPW + 16,), jnp.int32),
            pltpu.VMEM((PPW + 16,), jnp.int32),
            pltpu.VMEM((PPW + 16,), jnp.float32),
            pltpu.VMEM((PPW + 16,), jnp.float32),
            pltpu.VMEM((PPW + 16,), jnp.float32),
            pltpu.VMEM((CPAD,), jnp.float32),
            pltpu.VMEM((CPAD,), jnp.float32),
            pltpu.VMEM((CPAD,), jnp.int32),
            pltpu.VMEM((CPAD,), jnp.int32),
            pltpu.VMEM((CPAD,), jnp.int32),
        ],
        compiler_params=pltpu.CompilerParams(needs_layout_passes=False),
    )(_splat_body)


def _merge_body(a_ref, b_ref, c_ref, d_ref, o_ref):
    o_ref[...] = jnp.maximum(jnp.maximum(a_ref[...], b_ref[...]),
                             jnp.maximum(c_ref[...], d_ref[...]))


def _make_merge():
    def imap(k):
        return lambda b: (4 * b + k, 0, 0)

    return pl.pallas_call(
        _merge_body,
        grid=(BATCH,),
        in_specs=[pl.BlockSpec((1, 1, NPIX), imap(k)) for k in range(4)],
        out_specs=pl.BlockSpec((1, 1, NPIX), lambda b: (b, 0, 0)),
        out_shape=jax.ShapeDtypeStruct((BATCH, 1, NPIX), jnp.float32),
    )


_merge = _make_merge()


def kernel(data):
    d = data.reshape(TOT, 3).T
    d = jnp.pad(d, ((0, 5), (0, 0)))
    bi, bj, fi, fj, ft = _proj(d)
    bi = bi.reshape(TOT)
    bj = bj.reshape(TOT)
    fi = fi.reshape(TOT)
    fj = fj.reshape(TOT)
    ft = ft.reshape(TOT)
    copies = _make_splat()(bi, bj, fi, fj, ft,
                    jnp.asarray(_DIF_NP), jnp.asarray(_DJF_NP),
                    jnp.asarray(_DII_NP), jnp.asarray(_DJI_NP),
                    jnp.asarray(_DIDX_NP))
    c3 = copies.reshape(NWORK, 1, NPIX)
    img = _merge(c3, c3, c3, c3)
    return img.reshape(BATCH, 1, IMG, IMG)


# speedup vs baseline: 134.8464x; 1.1316x over previous
import functools
import math

import ml_dtypes
import numpy as np
import jax
import jax.numpy as jnp
from jax import lax
from jax.experimental import pallas as pl
from jax.experimental.pallas import tpu as pltpu
from jax.experimental.pallas import tpu_sc as plsc

IMG = 256
NPIX = IMG * IMG
BATCH = 8
NPTS = 2048
TOT = BATCH * NPTS
NWORK = 32
PPW = TOT // NWORK
WIN = 22


def _build_mat():
    def normalize(x):
        n = np.linalg.norm(x, axis=1, keepdims=True).astype(np.float32)
        return x / np.maximum(n, np.float32(1e-6))

    eyes = np.array([[-1.0, -1.0, -1.0]], np.float32)
    centers = np.zeros((1, 3), np.float32)
    ups = np.array([[0.0, 0.0, 1.0]], np.float32)
    zaxis = normalize(eyes - centers)
    xaxis = normalize(np.cross(ups, zaxis).astype(np.float32))
    yaxis = np.cross(zaxis, xaxis).astype(np.float32)
    translation = np.eye(4, dtype=np.float32)
    translation[:3, 3] = -eyes[0]
    orientation = np.eye(4, dtype=np.float32)
    orientation[0, :3] = xaxis[0]
    orientation[1, :3] = yaxis[0]
    orientation[2, :3] = zaxis[0]
    view = orientation @ translation
    z_near, z_far = np.float32(0.1), np.float32(10.0)
    proj = np.zeros((4, 4), np.float32)
    proj[0, 0] = 1.5
    proj[1, 1] = 1.5
    proj[2, 2] = -2.0 / (z_far - z_near)
    proj[2, 3] = (z_far + z_near) / (z_far - z_near)
    proj[3, 3] = 1.0
    return (proj @ view).astype(np.float32)


_MAT = _build_mat()
_MAT_BF = _MAT.astype(ml_dtypes.bfloat16).astype(np.float32)

_A2 = (math.pi / 20.0) ** 2
_COEFS = tuple(
    (-1.0) ** k * _A2 ** k / math.factorial(2 * k) for k in range(7)
)

_offs = np.arange(-10, 12)
_DI_g, _DJ_g = np.meshgrid(_offs, _offs, indexing="ij")
_mind = np.where(_offs <= 0, -_offs, _offs - 1)
_MI_g, _MJ_g = np.meshgrid(_mind, _mind, indexing="ij")
_keep = (_MI_g ** 2 + _MJ_g ** 2) <= 100
_DI = _DI_g[_keep]
_DJ = _DJ_g[_keep]
NCELLC = _DI.size
NCHUNK = (NCELLC + 15) // 16
CPAD = NCHUNK * 16
_pad = np.full(CPAD - NCELLC, 10000)
_DI = np.concatenate([_DI, _pad])
_DJ = np.concatenate([_DJ, _pad])
_DIF_NP = _DI.astype(np.float32)
_DJF_NP = _DJ.astype(np.float32)
_DII_NP = _DI.astype(np.int32)
_DJI_NP = _DJ.astype(np.int32)
_DIDX_NP = np.where(_DI >= 1000, 0, _DI * IMG + _DJ).astype(np.int32)


def _proj_body(d_ref, bi_ref, bj_ref, fi_ref, fj_ref, ft_ref):
    x = d_ref[0:1, :].astype(jnp.bfloat16).astype(jnp.float32)
    y = d_ref[1:2, :].astype(jnp.bfloat16).astype(jnp.float32)
    z = d_ref[2:3, :].astype(jnp.bfloat16).astype(jnp.float32)
    m = _MAT_BF
    pos_x = m[0, 0] * x + m[0, 1] * y + m[0, 2] * z + m[0, 3]
    pos_y = m[1, 0] * x + m[1, 1] * y + m[1, 2] * z + m[1, 3]
    pos_z = m[2, 0] * x + m[2, 1] * y + m[2, 2] * z + m[2, 3]
    pts_i = (-pos_y + 1.0) / 2.0 * (IMG - 1.0)
    pts_j = (pos_x + 1.0) / 2.0 * (IMG - 1.0)
    zmin = jnp.min(pos_z)
    zmax = jnp.max(pos_z)
    feat = 1.0 - (pos_z - zmin) / (zmax - zmin)
    base_i = jnp.floor(pts_i)
    base_j = jnp.floor(pts_j)
    fi_ref[...] = pts_i - base_i
    fj_ref[...] = pts_j - base_j
    ft_ref[...] = feat
    bi_ref[...] = base_i.astype(jnp.int32)
    bj_ref[...] = base_j.astype(jnp.int32)


_proj = pl.pallas_call(
    _proj_body,
    out_shape=[
        jax.ShapeDtypeStruct((1, TOT), jnp.int32),
        jax.ShapeDtypeStruct((1, TOT), jnp.int32),
        jax.ShapeDtypeStruct((1, TOT), jnp.float32),
        jax.ShapeDtypeStruct((1, TOT), jnp.float32),
        jax.ShapeDtypeStruct((1, TOT), jnp.float32),
    ],
)


def _splat_body(bi_h, bj_h, fi_h, fj_h, ft_h, dif_h, djf_h, dii_h, dji_h,
                didx_h, out_h, stg_h, img, bi, bj, fi, fj, ft,
                ibi, ibj, ifi, ifj, ift, bbi, bbj, bfi, bfj, bft,
                dif, djf, dii, dji, didx):
    cid = lax.axis_index("c")
    sid = lax.axis_index("s")
    wid = cid * 16 + sid
    base = wid * PPW

    pltpu.sync_copy(bi_h.at[pl.ds(base, PPW)], bi)
    pltpu.sync_copy(bj_h.at[pl.ds(base, PPW)], bj)
    pltpu.sync_copy(fi_h.at[pl.ds(base, PPW)], fi)
    pltpu.sync_copy(fj_h.at[pl.ds(base, PPW)], fj)
    pltpu.sync_copy(ft_h.at[pl.ds(base, PPW)], ft)
    pltpu.sync_copy(dif_h, dif)
    pltpu.sync_copy(djf_h, djf)
    pltpu.sync_copy(dii_h, dii)
    pltpu.sync_copy(dji_h, dji)
    pltpu.sync_copy(didx_h, didx)

    zeros16 = jnp.zeros((16,), jnp.float32)
    zidx = jnp.zeros((16,), jnp.int32)

    def zbody(k, c):
        b0 = k * 128
        for u in range(8):
            img[pl.ds(b0 + u * 16, 16)] = zeros16
        return c

    lax.fori_loop(0, NPIX // 128, zbody, 0)

    nivec = jnp.zeros((16,), jnp.int32)
    nbvec = jnp.zeros((16,), jnp.int32)
    for c in range(PPW // 16):
        s = c * 16
        bic = bi[pl.ds(s, 16)]
        bjc = bj[pl.ds(s, 16)]
        fic = fi[pl.ds(s, 16)]
        fjc = fj[pl.ds(s, 16)]
        ftc = ft[pl.ds(s, 16)]
        act = ((bic >= -10) & (bic <= IMG + 9)
               & (bjc >= -10) & (bjc <= IMG + 9))
        inter = ((bic >= 10) & (bic <= IMG - 12)
                 & (bjc >= 10) & (bjc <= IMG - 12))
        bord = act & (~inter)
        ini = inter.astype(jnp.int32)
        boi = bord.astype(jnp.int32)
        dsti = nivec + plsc.cumsum(ini) - ini
        dstb = nbvec + plsc.cumsum(boi) - boi
        plsc.store_scatter(ibi, [dsti], bic, mask=inter)
        plsc.store_scatter(ibj, [dsti], bjc, mask=inter)
        plsc.store_scatter(ifi, [dsti], fic, mask=inter)
        plsc.store_scatter(ifj, [dsti], fjc, mask=inter)
        plsc.store_scatter(ift, [dsti], ftc, mask=inter)
        plsc.store_scatter(bbi, [dstb], bic, mask=bord)
        plsc.store_scatter(bbj, [dstb], bjc, mask=bord)
        plsc.store_scatter(bfi, [dstb], fic, mask=bord)
        plsc.store_scatter(bfj, [dstb], fjc, mask=bord)
        plsc.store_scatter(bft, [dstb], ftc, mask=bord)
        nivec = nivec + plsc.all_reduce_population_count(inter)
        nbvec = nbvec + plsc.all_reduce_population_count(bord)
    ni = jnp.max(nivec)
    nb = jnp.max(nbvec)

    def ibody(p, carry):
        bip = ibi[pl.ds(p, 16)][zidx]
        bjp = ibj[pl.ds(p, 16)][zidx]
        fip = ifi[pl.ds(p, 16)][zidx]
        fjp = ifj[pl.ds(p, 16)][zidx]
        ftp = ift[pl.ds(p, 16)][zidx]
        pbase = (bip << 8) + bjp

        @plsc.parallel_loop(0, CPAD, 16)
        def chunk(s):
            ai = dif[pl.ds(s, 16)] - fip
            aj = djf[pl.ds(s, 16)] - fjp
            d2 = ai * ai + aj * aj
            w = jnp.float32(_COEFS[-1])
            for cf in _COEFS[-2::-1]:
                w = w * d2 + jnp.float32(cf)
            val = w * ftp
            idx = pbase + didx[pl.ds(s, 16)]
            msk = d2 <= 100.0
            old = plsc.load_gather(img, [idx])
            plsc.store_scatter(img, [idx], jnp.maximum(old, val), mask=msk)

        return carry

    lax.fori_loop(0, ni, ibody, 0)

    def bbody(p, carry):
        bip = bbi[pl.ds(p, 16)][zidx]
        bjp = bbj[pl.ds(p, 16)][zidx]
        fip = bfi[pl.ds(p, 16)][zidx]
        fjp = bfj[pl.ds(p, 16)][zidx]
        ftp = bft[pl.ds(p, 16)][zidx]

        @plsc.parallel_loop(0, CPAD, 16)
        def chunk(s):
            ai = dif[pl.ds(s, 16)] - fip
            aj = djf[pl.ds(s, 16)] - fjp
            d2 = ai * ai + aj * aj
            w = jnp.float32(_COEFS[-1])
            for cf in _COEFS[-2::-1]:
                w = w * d2 + jnp.float32(cf)
            val = w * ftp
            ii = bip + dii[pl.ds(s, 16)]
            jj = bjp + dji[pl.ds(s, 16)]
            msk = ((ii >= 0) & (ii <= IMG - 1) & (jj >= 0) & (jj <= IMG - 1)
                   & (d2 <= 100.0))
            idx = ((ii << 8) + jj) & (NPIX - 1)
            old = plsc.load_gather(img, [idx])
            plsc.store_scatter(img, [idx], jnp.maximum(old, val), mask=msk)

        return carry

    lax.fori_loop(0, nb, bbody, 0)

    pltpu.sync_copy(img, stg_h.at[wid])
    plsc.subcore_barrier()
    QS = NPIX // 4
    q = sid % 4
    g = sid // 4
    b = cid * 4 + g
    qoff = q * QS
    for k in range(4):
        pltpu.sync_copy(stg_h.at[g * 4 + k + cid * 16, pl.ds(qoff, QS)],
                        img.at[pl.ds(k * QS, QS)])

    @plsc.parallel_loop(0, QS, 16)
    def mbody(s):
        a = jnp.maximum(img[pl.ds(s, 16)], img[pl.ds(QS + s, 16)])
        c2 = jnp.maximum(img[pl.ds(2 * QS + s, 16)],
                         img[pl.ds(3 * QS + s, 16)])
        img[pl.ds(s, 16)] = jnp.maximum(a, c2)

    pltpu.sync_copy(img.at[pl.ds(0, QS)], out_h.at[b, pl.ds(qoff, QS)])


@functools.lru_cache(maxsize=None)
def _make_splat():
    return functools.partial(
        pl.kernel,
        out_type=[jax.ShapeDtypeStruct((BATCH, NPIX), jnp.float32),
                  jax.ShapeDtypeStruct((NWORK, NPIX), jnp.float32)],
        mesh=plsc.VectorSubcoreMesh(core_axis_name="c", subcore_axis_name="s",
                                    num_cores=2, num_subcores=16),
        scratch_types=[
            pltpu.VMEM((NPIX,), jnp.float32),
            pltpu.VMEM((PPW,), jnp.int32),
            pltpu.VMEM((PPW,), jnp.int32),
            pltpu.VMEM((PPW,), jnp.float32),
            pltpu.VMEM((PPW,), jnp.float32),
            pltpu.VMEM((PPW,), jnp.float32),
            pltpu.VMEM((PPW + 16,), jnp.int32),
            pltpu.VMEM((PPW + 16,), jnp.int32),
            pltpu.VMEM((PPW + 16,), jnp.float32),
            pltpu.VMEM((PPW + 16,), jnp.float32),
            pltpu.VMEM((PPW + 16,), jnp.float32),
            pltpu.VMEM((PPW + 16,), jnp.int32),
            pltpu.VMEM((PPW + 16,), jnp.int32),
            pltpu.VMEM((PPW + 16,), jnp.float32),
            pltpu.VMEM((PPW + 16,), jnp.float32),
            pltpu.VMEM((PPW + 16,), jnp.float32),
            pltpu.VMEM((CPAD,), jnp.float32),
            pltpu.VMEM((CPAD,), jnp.float32),
            pltpu.VMEM((CPAD,), jnp.int32),
            pltpu.VMEM((CPAD,), jnp.int32),
            pltpu.VMEM((CPAD,), jnp.int32),
        ],
        compiler_params=pltpu.CompilerParams(needs_layout_passes=False),
    )(_splat_body)


def kernel(data):
    d = data.reshape(TOT, 3).T
    d = jnp.pad(d, ((0, 5), (0, 0)))
    bi, bj, fi, fj, ft = _proj(d)
    bi = bi.reshape(TOT)
    bj = bj.reshape(TOT)
    fi = fi.reshape(TOT)
    fj = fj.reshape(TOT)
    ft = ft.reshape(TOT)
    img, _ = _make_splat()(bi, bj, fi, fj, ft,
                           jnp.asarray(_DIF_NP), jnp.asarray(_DJF_NP),
                           jnp.asarray(_DII_NP), jnp.asarray(_DJI_NP),
                           jnp.asarray(_DIDX_NP))
    return img.reshape(BATCH, 1, IMG, IMG)


# speedup vs baseline: 136.7862x; 1.0144x over previous
import functools
import math

import ml_dtypes
import numpy as np
import jax
import jax.numpy as jnp
from jax import lax
from jax.experimental import pallas as pl
from jax.experimental.pallas import tpu as pltpu
from jax.experimental.pallas import tpu_sc as plsc

IMG = 256
NPIX = IMG * IMG
BATCH = 8
NPTS = 2048
TOT = BATCH * NPTS
NWORK = 32
PPW = TOT // NWORK
WIN = 22


def _build_mat():
    def normalize(x):
        n = np.linalg.norm(x, axis=1, keepdims=True).astype(np.float32)
        return x / np.maximum(n, np.float32(1e-6))

    eyes = np.array([[-1.0, -1.0, -1.0]], np.float32)
    centers = np.zeros((1, 3), np.float32)
    ups = np.array([[0.0, 0.0, 1.0]], np.float32)
    zaxis = normalize(eyes - centers)
    xaxis = normalize(np.cross(ups, zaxis).astype(np.float32))
    yaxis = np.cross(zaxis, xaxis).astype(np.float32)
    translation = np.eye(4, dtype=np.float32)
    translation[:3, 3] = -eyes[0]
    orientation = np.eye(4, dtype=np.float32)
    orientation[0, :3] = xaxis[0]
    orientation[1, :3] = yaxis[0]
    orientation[2, :3] = zaxis[0]
    view = orientation @ translation
    z_near, z_far = np.float32(0.1), np.float32(10.0)
    proj = np.zeros((4, 4), np.float32)
    proj[0, 0] = 1.5
    proj[1, 1] = 1.5
    proj[2, 2] = -2.0 / (z_far - z_near)
    proj[2, 3] = (z_far + z_near) / (z_far - z_near)
    proj[3, 3] = 1.0
    return (proj @ view).astype(np.float32)


_MAT = _build_mat()
_MAT_BF = _MAT.astype(ml_dtypes.bfloat16).astype(np.float32)

_A2 = (math.pi / 20.0) ** 2
_COEFS = tuple(
    (-1.0) ** k * _A2 ** k / math.factorial(2 * k) for k in range(7)
)

_offs = np.arange(-10, 12)
_DI_g, _DJ_g = np.meshgrid(_offs, _offs, indexing="ij")
_mind = np.where(_offs <= 0, -_offs, _offs - 1)
_MI_g, _MJ_g = np.meshgrid(_mind, _mind, indexing="ij")
_keep = (_MI_g ** 2 + _MJ_g ** 2) <= 100
_DI = _DI_g[_keep]
_DJ = _DJ_g[_keep]
NCELLC = _DI.size
NCHUNK = (NCELLC + 15) // 16
CPAD = NCHUNK * 16
_pad = np.full(CPAD - NCELLC, 10000)
_DI = np.concatenate([_DI, _pad])
_DJ = np.concatenate([_DJ, _pad])
_DIF_NP = _DI.astype(np.float32)
_DJF_NP = _DJ.astype(np.float32)
_DII_NP = _DI.astype(np.int32)
_DJI_NP = _DJ.astype(np.int32)
_DIDX_NP = np.where(_DI >= 1000, 0, _DI * IMG + _DJ).astype(np.int32)


def _proj_body(d_ref, bi_ref, bj_ref, fi_ref, fj_ref, ft_ref):
    x = d_ref[0:1, :].astype(jnp.bfloat16).astype(jnp.float32)
    y = d_ref[1:2, :].astype(jnp.bfloat16).astype(jnp.float32)
    z = d_ref[2:3, :].astype(jnp.bfloat16).astype(jnp.float32)
    m = _MAT_BF
    pos_x = m[0, 0] * x + m[0, 1] * y + m[0, 2] * z + m[0, 3]
    pos_y = m[1, 0] * x + m[1, 1] * y + m[1, 2] * z + m[1, 3]
    pos_z = m[2, 0] * x + m[2, 1] * y + m[2, 2] * z + m[2, 3]
    pts_i = (-pos_y + 1.0) / 2.0 * (IMG - 1.0)
    pts_j = (pos_x + 1.0) / 2.0 * (IMG - 1.0)
    zmin = jnp.min(pos_z)
    zmax = jnp.max(pos_z)
    feat = 1.0 - (pos_z - zmin) / (zmax - zmin)
    base_i = jnp.floor(pts_i)
    base_j = jnp.floor(pts_j)
    fi_ref[...] = pts_i - base_i
    fj_ref[...] = pts_j - base_j
    ft_ref[...] = feat
    bi_ref[...] = base_i.astype(jnp.int32)
    bj_ref[...] = base_j.astype(jnp.int32)


_proj = pl.pallas_call(
    _proj_body,
    out_shape=[
        jax.ShapeDtypeStruct((1, TOT), jnp.int32),
        jax.ShapeDtypeStruct((1, TOT), jnp.int32),
        jax.ShapeDtypeStruct((1, TOT), jnp.float32),
        jax.ShapeDtypeStruct((1, TOT), jnp.float32),
        jax.ShapeDtypeStruct((1, TOT), jnp.float32),
    ],
)


def _splat_body(bi_h, bj_h, fi_h, fj_h, ft_h, dif_h, djf_h, dii_h, dji_h,
                didx_h, out_h, stg_h, img, bi, bj, fi, fj, ft,
                ibi, ibj, ifi, ifj, ift, bbi, bbj, bfi, bfj, bft,
                dif, djf, dii, dji, didx):
    cid = lax.axis_index("c")
    sid = lax.axis_index("s")
    wid = cid * 16 + sid
    base = wid * PPW

    pltpu.sync_copy(bi_h.at[pl.ds(base, PPW)], bi)
    pltpu.sync_copy(bj_h.at[pl.ds(base, PPW)], bj)
    pltpu.sync_copy(fi_h.at[pl.ds(base, PPW)], fi)
    pltpu.sync_copy(fj_h.at[pl.ds(base, PPW)], fj)
    pltpu.sync_copy(ft_h.at[pl.ds(base, PPW)], ft)
    pltpu.sync_copy(dif_h, dif)
    pltpu.sync_copy(djf_h, djf)
    pltpu.sync_copy(dii_h, dii)
    pltpu.sync_copy(dji_h, dji)
    pltpu.sync_copy(didx_h, didx)

    zeros16 = jnp.zeros((16,), jnp.float32)
    zidx = jnp.zeros((16,), jnp.int32)

    @plsc.parallel_loop(0, NPIX, 16, unroll=8)
    def zbody(s):
        img[pl.ds(s, 16)] = zeros16

    nivec = jnp.zeros((16,), jnp.int32)
    nbvec = jnp.zeros((16,), jnp.int32)
    for c in range(PPW // 16):
        s = c * 16
        bic = bi[pl.ds(s, 16)]
        bjc = bj[pl.ds(s, 16)]
        fic = fi[pl.ds(s, 16)]
        fjc = fj[pl.ds(s, 16)]
        ftc = ft[pl.ds(s, 16)]
        act = ((bic >= -10) & (bic <= IMG + 9)
               & (bjc >= -10) & (bjc <= IMG + 9))
        inter = ((bic >= 10) & (bic <= IMG - 12)
                 & (bjc >= 10) & (bjc <= IMG - 12))
        bord = act & (~inter)
        ini = inter.astype(jnp.int32)
        boi = bord.astype(jnp.int32)
        dsti = nivec + plsc.cumsum(ini) - ini
        dstb = nbvec + plsc.cumsum(boi) - boi
        plsc.store_scatter(ibi, [dsti], bic, mask=inter)
        plsc.store_scatter(ibj, [dsti], bjc, mask=inter)
        plsc.store_scatter(ifi, [dsti], fic, mask=inter)
        plsc.store_scatter(ifj, [dsti], fjc, mask=inter)
        plsc.store_scatter(ift, [dsti], ftc, mask=inter)
        plsc.store_scatter(bbi, [dstb], bic, mask=bord)
        plsc.store_scatter(bbj, [dstb], bjc, mask=bord)
        plsc.store_scatter(bfi, [dstb], fic, mask=bord)
        plsc.store_scatter(bfj, [dstb], fjc, mask=bord)
        plsc.store_scatter(bft, [dstb], ftc, mask=bord)
        nivec = nivec + plsc.all_reduce_population_count(inter)
        nbvec = nbvec + plsc.all_reduce_population_count(bord)
    ni = jnp.max(nivec)
    nb = jnp.max(nbvec)

    def ibody(p, carry):
        bip = ibi[pl.ds(p, 16)][zidx]
        bjp = ibj[pl.ds(p, 16)][zidx]
        fip = ifi[pl.ds(p, 16)][zidx]
        fjp = ifj[pl.ds(p, 16)][zidx]
        ftp = ift[pl.ds(p, 16)][zidx]
        pbase = (bip << 8) + bjp

        @plsc.parallel_loop(0, CPAD, 16, unroll=2)
        def chunk(s):
            ai = dif[pl.ds(s, 16)] - fip
            aj = djf[pl.ds(s, 16)] - fjp
            d2 = ai * ai + aj * aj
            w = jnp.float32(_COEFS[-1])
            for cf in _COEFS[-2::-1]:
                w = w * d2 + jnp.float32(cf)
            val = w * ftp
            idx = pbase + didx[pl.ds(s, 16)]
            msk = d2 <= 100.0
            old = plsc.load_gather(img, [idx])
            plsc.store_scatter(img, [idx], jnp.maximum(old, val), mask=msk)

        return carry

    lax.fori_loop(0, ni, ibody, 0)

    def bbody(p, carry):
        bip = bbi[pl.ds(p, 16)][zidx]
        bjp = bbj[pl.ds(p, 16)][zidx]
        fip = bfi[pl.ds(p, 16)][zidx]
        fjp = bfj[pl.ds(p, 16)][zidx]
        ftp = bft[pl.ds(p, 16)][zidx]

        @plsc.parallel_loop(0, CPAD, 16, unroll=2)
        def chunk(s):
            ai = dif[pl.ds(s, 16)] - fip
            aj = djf[pl.ds(s, 16)] - fjp
            d2 = ai * ai + aj * aj
            w = jnp.float32(_COEFS[-1])
            for cf in _COEFS[-2::-1]:
                w = w * d2 + jnp.float32(cf)
            val = w * ftp
            ii = bip + dii[pl.ds(s, 16)]
            jj = bjp + dji[pl.ds(s, 16)]
            msk = ((ii >= 0) & (ii <= IMG - 1) & (jj >= 0) & (jj <= IMG - 1)
                   & (d2 <= 100.0))
            idx = ((ii << 8) + jj) & (NPIX - 1)
            old = plsc.load_gather(img, [idx])
            plsc.store_scatter(img, [idx], jnp.maximum(old, val), mask=msk)

        return carry

    lax.fori_loop(0, nb, bbody, 0)

    pltpu.sync_copy(img, stg_h.at[wid])
    plsc.subcore_barrier()
    QS = NPIX // 4
    q = sid % 4
    g = sid // 4
    b = cid * 4 + g
    qoff = q * QS
    for k in range(4):
        pltpu.sync_copy(stg_h.at[g * 4 + k + cid * 16, pl.ds(qoff, QS)],
                        img.at[pl.ds(k * QS, QS)])

    @plsc.parallel_loop(0, QS, 16, unroll=4)
    def mbody(s):
        a = jnp.maximum(img[pl.ds(s, 16)], img[pl.ds(QS + s, 16)])
        c2 = jnp.maximum(img[pl.ds(2 * QS + s, 16)],
                         img[pl.ds(3 * QS + s, 16)])
        img[pl.ds(s, 16)] = jnp.maximum(a, c2)

    pltpu.sync_copy(img.at[pl.ds(0, QS)], out_h.at[b, pl.ds(qoff, QS)])


@functools.lru_cache(maxsize=None)
def _make_splat():
    return functools.partial(
        pl.kernel,
        out_type=[jax.ShapeDtypeStruct((BATCH, NPIX), jnp.float32),
                  jax.ShapeDtypeStruct((NWORK, NPIX), jnp.float32)],
        mesh=plsc.VectorSubcoreMesh(core_axis_name="c", subcore_axis_name="s",
                                    num_cores=2, num_subcores=16),
        scratch_types=[
            pltpu.VMEM((NPIX,), jnp.float32),
            pltpu.VMEM((PPW,), jnp.int32),
            pltpu.VMEM((PPW,), jnp.int32),
            pltpu.VMEM((PPW,), jnp.float32),
            pltpu.VMEM((PPW,), jnp.float32),
            pltpu.VMEM((PPW,), jnp.float32),
            pltpu.VMEM((PPW + 16,), jnp.int32),
            pltpu.VMEM((PPW + 16,), jnp.int32),
            pltpu.VMEM((PPW + 16,), jnp.float32),
            pltpu.VMEM((PPW + 16,), jnp.float32),
            pltpu.VMEM((PPW + 16,), jnp.float32),
            pltpu.VMEM((PPW + 16,), jnp.int32),
            pltpu.VMEM((PPW + 16,), jnp.int32),
            pltpu.VMEM((PPW + 16,), jnp.float32),
            pltpu.VMEM((PPW + 16,), jnp.float32),
            pltpu.VMEM((PPW + 16,), jnp.float32),
            pltpu.VMEM((CPAD,), jnp.float32),
            pltpu.VMEM((CPAD,), jnp.float32),
            pltpu.VMEM((CPAD,), jnp.int32),
            pltpu.VMEM((CPAD,), jnp.int32),
            pltpu.VMEM((CPAD,), jnp.int32),
        ],
        compiler_params=pltpu.CompilerParams(needs_layout_passes=False),
    )(_splat_body)


def kernel(data):
    d = data.reshape(TOT, 3).T
    d = jnp.pad(d, ((0, 5), (0, 0)))
    bi, bj, fi, fj, ft = _proj(d)
    bi = bi.reshape(TOT)
    bj = bj.reshape(TOT)
    fi = fi.reshape(TOT)
    fj = fj.reshape(TOT)
    ft = ft.reshape(TOT)
    img, _ = _make_splat()(bi, bj, fi, fj, ft,
                           jnp.asarray(_DIF_NP), jnp.asarray(_DJF_NP),
                           jnp.asarray(_DII_NP), jnp.asarray(_DJI_NP),
                           jnp.asarray(_DIDX_NP))
    return img.reshape(BATCH, 1, IMG, IMG)
